# baseline (speedup 1.0000x reference)
import jax
import jax.numpy as jnp
from jax.experimental import pallas as pl
from jax.experimental.pallas import tpu as pltpu

_K = 2
_H = 16
_T = 2048
_S = 2048
_BT = 1024
_NT = _T // _BT
_TAIL = 8


def _merge_rows(last):
    col = jax.lax.broadcasted_iota(jnp.int32, (_H, _S), 1)
    row = jax.lax.broadcasted_iota(jnp.int32, (_H, 1), 0)
    maxv = jnp.max(last, axis=1, keepdims=True)
    cand = jnp.min(jnp.where(last == maxv, col, _S), axis=1, keepdims=True)
    onehot = col == cand
    hist = jnp.sum(onehot.astype(jnp.int32), axis=0, keepdims=True)
    cph = jnp.sum(jnp.where(onehot, hist, 0), axis=1, keepdims=True)
    maxc = jnp.max(hist)
    mask = cph == maxc
    do_merge = maxc <= _K
    nmax = jnp.sum(mask.astype(jnp.int32))
    pos = jnp.where(
        nmax <= 4, jnp.int32(0),
        jnp.where(nmax <= 8, jnp.int32(4),
                  jnp.where(nmax <= 12, jnp.int32(5), jnp.int32(12))))
    key = jnp.where(mask, cand * _H + row, _S * _H + row)
    big = jnp.int32(2 * _S * _H)

    def body(_, carry):
        cur, _m = carry
        m = jnp.min(cur)
        return jnp.where(cur == m, big, cur), m

    _, mkey = jax.lax.fori_loop(0, pos + 1, body, (key, jnp.int32(0)))
    shead = jnp.sum(jnp.where(key == mkey, row, 0))
    src = jnp.sum(jnp.where(row == shead, last, 0.0), axis=0, keepdims=True)
    return jnp.where(jnp.logical_and(do_merge, mask), src, last)


_NB = 8
_BC = 512
_BULK = _T - _TAIL
_CHUNKS = [(h, off, min(_BC, _BULK - off)) for h in range(_H)
           for off in range(0, _BULK, _BC)]


def _relay_kernel(x_hbm, o_hbm, bufs, tail, rsem, wsem, tsem):
    tread = pltpu.make_async_copy(
        x_hbm.at[:, :, pl.ds(_BULK, _TAIL), :], tail, tsem)
    tread.start()

    def read(c):
        h, off, sz = _CHUNKS[c]
        return pltpu.make_async_copy(
            x_hbm.at[0, h, pl.ds(off, sz), :],
            bufs.at[c % _NB, pl.ds(0, sz), :], rsem.at[c % _NB])

    def write(c):
        h, off, sz = _CHUNKS[c]
        return pltpu.make_async_copy(
            bufs.at[c % _NB, pl.ds(0, sz), :],
            o_hbm.at[0, h, pl.ds(off, sz), :], wsem.at[c % _NB])

    nc = len(_CHUNKS)
    for c in range(_NB):
        read(c).start()

    tread.wait()
    tail[0, :, _TAIL - 1, :] = _merge_rows(tail[0, :, _TAIL - 1, :])
    twrite = pltpu.make_async_copy(
        tail, o_hbm.at[:, :, pl.ds(_BULK, _TAIL), :], tsem)
    twrite.start()

    for c in range(nc):
        read(c).wait()
        write(c).start()
        if c + _NB < nc:
            write(c).wait()
            read(c + _NB).start()
    twrite.wait()
    for c in range(nc - _NB, nc):
        write(c).wait()


def kernel(attention_weight):
    out = pl.pallas_call(
        _relay_kernel,
        in_specs=[pl.BlockSpec(memory_space=pl.ANY)],
        out_specs=pl.BlockSpec(memory_space=pl.ANY),
        out_shape=jax.ShapeDtypeStruct((1, _H, _T, _S), jnp.float32),
        scratch_shapes=[
            pltpu.VMEM((_NB, _BC, _S), jnp.float32),
            pltpu.VMEM((1, _H, _TAIL, _S), jnp.float32),
            pltpu.SemaphoreType.DMA((_NB,)),
            pltpu.SemaphoreType.DMA((_NB,)),
            pltpu.SemaphoreType.DMA,
        ],
    )(attention_weight)
    return out


# speedup vs baseline: 1.0319x; 1.0319x over previous
import jax
import jax.numpy as jnp
from jax.experimental import pallas as pl
from jax.experimental.pallas import tpu as pltpu

_K = 2
_H = 16
_T = 2048
_S = 2048
_BT = 1024
_NT = _T // _BT
_TAIL = 8


def _merge_rows(last):
    col = jax.lax.broadcasted_iota(jnp.int32, (_H, _S), 1)
    row = jax.lax.broadcasted_iota(jnp.int32, (_H, 1), 0)
    maxv = jnp.max(last, axis=1, keepdims=True)
    cand = jnp.min(jnp.where(last == maxv, col, _S), axis=1, keepdims=True)
    onehot = col == cand
    hist = jnp.sum(onehot.astype(jnp.int32), axis=0, keepdims=True)
    cph = jnp.sum(jnp.where(onehot, hist, 0), axis=1, keepdims=True)
    maxc = jnp.max(hist)
    mask = cph == maxc
    do_merge = maxc <= _K
    nmax = jnp.sum(mask.astype(jnp.int32))
    pos = jnp.where(
        nmax <= 4, jnp.int32(0),
        jnp.where(nmax <= 8, jnp.int32(4),
                  jnp.where(nmax <= 12, jnp.int32(5), jnp.int32(12))))
    key = jnp.where(mask, cand * _H + row, _S * _H + row)
    big = jnp.int32(2 * _S * _H)

    def body(_, carry):
        cur, _m = carry
        m = jnp.min(cur)
        return jnp.where(cur == m, big, cur), m

    _, mkey = jax.lax.fori_loop(0, pos + 1, body, (key, jnp.int32(0)))
    shead = jnp.sum(jnp.where(key == mkey, row, 0))
    src = jnp.sum(jnp.where(row == shead, last, 0.0), axis=0, keepdims=True)
    return jnp.where(jnp.logical_and(do_merge, mask), src, last)


_NB = 6
_BC = 1024
_BULK = _T - _TAIL
_CHUNKS = [(h, off, min(_BC, _BULK - off)) for h in range(_H)
           for off in range(0, _BULK, _BC)]


def _relay_kernel(x_hbm, o_hbm, bufs, tail, rsem, wsem, tsem):
    tread = pltpu.make_async_copy(
        x_hbm.at[:, :, pl.ds(_BULK, _TAIL), :], tail, tsem)
    tread.start()

    def read(c):
        h, off, sz = _CHUNKS[c]
        return pltpu.make_async_copy(
            x_hbm.at[0, h, pl.ds(off, sz), :],
            bufs.at[c % _NB, pl.ds(0, sz), :], rsem.at[c % _NB])

    def write(c):
        h, off, sz = _CHUNKS[c]
        return pltpu.make_async_copy(
            bufs.at[c % _NB, pl.ds(0, sz), :],
            o_hbm.at[0, h, pl.ds(off, sz), :], wsem.at[c % _NB])

    nc = len(_CHUNKS)
    for c in range(_NB):
        read(c).start()

    tread.wait()
    tail[0, :, _TAIL - 1, :] = _merge_rows(tail[0, :, _TAIL - 1, :])
    twrite = pltpu.make_async_copy(
        tail, o_hbm.at[:, :, pl.ds(_BULK, _TAIL), :], tsem)
    twrite.start()

    for c in range(nc):
        read(c).wait()
        write(c).start()
        if c + _NB < nc:
            write(c).wait()
            read(c + _NB).start()
    twrite.wait()
    for c in range(nc - _NB, nc):
        write(c).wait()


def kernel(attention_weight):
    out = pl.pallas_call(
        _relay_kernel,
        in_specs=[pl.BlockSpec(memory_space=pl.ANY)],
        out_specs=pl.BlockSpec(memory_space=pl.ANY),
        out_shape=jax.ShapeDtypeStruct((1, _H, _T, _S), jnp.float32),
        scratch_shapes=[
            pltpu.VMEM((_NB, _BC, _S), jnp.float32),
            pltpu.VMEM((1, _H, _TAIL, _S), jnp.float32),
            pltpu.SemaphoreType.DMA((_NB,)),
            pltpu.SemaphoreType.DMA((_NB,)),
            pltpu.SemaphoreType.DMA,
        ],
    )(attention_weight)
    return out
